# Initial kernel scaffold; baseline (speedup 1.0000x reference)
import jax
import jax.numpy as jnp
from jax.experimental import pallas as pl


def kernel(x, Wqkv, Wo, Wr, W1, W2, g1, g2):
    raise NotImplementedError("write your pallas kernel here")


# speedup vs baseline: 1.1710x; 1.1710x over previous
import math

import jax
import jax.numpy as jnp
from jax import lax
from jax.experimental import pallas as pl
from jax.experimental.pallas import tpu as pltpu

_N_HEADS = 16
_BASE = 10000.0
_TOPK = 2
_CAP_FACTOR = 1.25
_EPS = 1e-6
_NE = 8
_D = 1024
_F = 4096
_S = 2048
_HD = _D // _N_HEADS
_HALF = _HD // 2
_DH = _D // 2
_CAP = max(math.ceil(_S * _TOPK * _CAP_FACTOR / _NE), 1)

_BT = 256
_NTB = _S // _BT
_HPB = 4
_NHB = _N_HEADS // _HPB
_BF = 512
_NFB = _F // _BF


def _nmap(*axes):
    def m(*pids):
        return tuple(pids[a] if isinstance(a, int) else 0 for a in axes)
    return m


def _k1_body(x_ref, g1_ref, w_ref, c_ref, s_ref,
             q0_ref, q1_ref, k0_ref, k1_ref, v_ref):
    x = x_ref[...]
    rms = lax.rsqrt(jnp.mean(x * x, axis=-1, keepdims=True) + _EPS)
    h = x * rms * g1_ref[...]
    qkv = lax.dot_general(h, w_ref[...], (((1,), (1,)), ((), ())),
                          preferred_element_type=jnp.float32)
    c = c_ref[...]
    s = s_ref[...]
    q0 = qkv[:, 0:_DH]
    q1 = qkv[:, _DH:_D]
    k0 = qkv[:, _D:_D + _DH]
    k1 = qkv[:, _D + _DH:2 * _D]
    q0_ref[...] = q0 * c - q1 * s
    q1_ref[...] = q0 * s + q1 * c
    k0_ref[...] = k0 * c - k1 * s
    k1_ref[...] = k0 * s + k1 * c
    v_ref[...] = qkv[:, 2 * _D:3 * _D]


def _k1(xf, g1r, wp, cc, ss):
    o = jax.ShapeDtypeStruct
    return pl.pallas_call(
        _k1_body,
        grid=(_NTB,),
        in_specs=[
            pl.BlockSpec((_BT, _D), _nmap(0, None)),
            pl.BlockSpec((1, _D), _nmap(None, None)),
            pl.BlockSpec((3 * _D, _D), _nmap(None, None)),
            pl.BlockSpec((_BT, _DH), _nmap(0, None)),
            pl.BlockSpec((_BT, _DH), _nmap(0, None)),
        ],
        out_specs=[pl.BlockSpec((_BT, _DH), _nmap(0, None))] * 4
        + [pl.BlockSpec((_BT, _D), _nmap(0, None))],
        out_shape=[o((_S, _DH), jnp.float32)] * 4 + [o((_S, _D), jnp.float32)],
    )(xf, g1r, wp, cc, ss)


def _k2_body(q0_ref, q1_ref, k0_ref, k1_ref, v_ref, y_ref):
    qb = pl.program_id(1)
    q0 = q0_ref[...]
    q1 = q1_ref[...]
    k0 = k0_ref[...]
    k1 = k1_ref[...]
    v = v_ref[...]
    row = qb * _BT + lax.broadcasted_iota(jnp.int32, (_BT, 1), 0)
    col = lax.broadcasted_iota(jnp.int32, (1, _S), 1)
    mask = col <= row
    dnt = (((1,), (1,)), ((), ()))
    dn = (((1,), (0,)), ((), ()))
    outs = []
    for hh in range(_HPB):
        hs = slice(hh * _HALF, (hh + 1) * _HALF)
        sc = (lax.dot_general(q0[:, hs], k0[:, hs], dnt,
                              preferred_element_type=jnp.float32)
              + lax.dot_general(q1[:, hs], k1[:, hs], dnt,
                                preferred_element_type=jnp.float32))
        sc = sc * (1.0 / math.sqrt(_HD))
        sc = jnp.where(mask, sc, -1e30)
        m = jnp.max(sc, axis=1, keepdims=True)
        p = jnp.exp(sc - m)
        p = p / jnp.sum(p, axis=1, keepdims=True)
        outs.append(lax.dot_general(p, v[:, hh * _HD:(hh + 1) * _HD], dn,
                                    preferred_element_type=jnp.float32))
    y_ref[...] = jnp.concatenate(outs, axis=1)


def _k2(q0, q1, k0, k1, v):
    hw = _HPB * _HALF
    vw = _HPB * _HD
    return pl.pallas_call(
        _k2_body,
        grid=(_NHB, _NTB),
        in_specs=[
            pl.BlockSpec((_BT, hw), _nmap(1, 0)),
            pl.BlockSpec((_BT, hw), _nmap(1, 0)),
            pl.BlockSpec((_S, hw), _nmap(None, 0)),
            pl.BlockSpec((_S, hw), _nmap(None, 0)),
            pl.BlockSpec((_S, vw), _nmap(None, 0)),
        ],
        out_specs=pl.BlockSpec((_BT, vw), _nmap(1, 0)),
        out_shape=jax.ShapeDtypeStruct((_S, _D), jnp.float32),
    )(q0, q1, k0, k1, v)


def _k3_body(x_ref, y_ref, wo_ref, g2_ref, wr_ref,
             x2_ref, h2_ref, gates_ref):
    x2 = x_ref[...] + lax.dot_general(
        y_ref[...], wo_ref[...], (((1,), (1,)), ((), ())),
        preferred_element_type=jnp.float32)
    x2_ref[...] = x2
    rms = lax.rsqrt(jnp.mean(x2 * x2, axis=-1, keepdims=True) + _EPS)
    h2 = x2 * rms * g2_ref[...]
    h2_ref[...] = h2
    logits = lax.dot_general(h2, wr_ref[...], (((1,), (1,)), ((), ())),
                             preferred_element_type=jnp.float32)
    lm = jnp.max(logits, axis=1, keepdims=True)
    el = jnp.exp(logits - lm)
    probs = el / jnp.sum(el, axis=1, keepdims=True)
    iota = lax.broadcasted_iota(jnp.int32, (_BT, _NE), 1)
    m1 = jnp.max(probs, axis=1, keepdims=True)
    e1 = jnp.min(jnp.where(probs == m1, iota, _NE), axis=1, keepdims=True)
    pm = jnp.where(iota == e1, -1.0, probs)
    m2 = jnp.max(pm, axis=1, keepdims=True)
    e2 = jnp.min(jnp.where(pm == m2, iota, _NE), axis=1, keepdims=True)
    gates_ref[...] = (jnp.where(iota == e1, m1, 0.0)
                      + jnp.where(iota == e2, m2, 0.0))


def _k3(xf, y, wo, g2r, wr):
    o = jax.ShapeDtypeStruct
    return pl.pallas_call(
        _k3_body,
        grid=(_NTB,),
        in_specs=[
            pl.BlockSpec((_BT, _D), _nmap(0, None)),
            pl.BlockSpec((_BT, _D), _nmap(0, None)),
            pl.BlockSpec((_D, _D), _nmap(None, None)),
            pl.BlockSpec((1, _D), _nmap(None, None)),
            pl.BlockSpec((_NE, _D), _nmap(None, None)),
        ],
        out_specs=[
            pl.BlockSpec((_BT, _D), _nmap(0, None)),
            pl.BlockSpec((_BT, _D), _nmap(0, None)),
            pl.BlockSpec((_BT, _NE), _nmap(0, None)),
        ],
        out_shape=[o((_S, _D), jnp.float32), o((_S, _D), jnp.float32),
                   o((_S, _NE), jnp.float32)],
    )(xf, y, wo, g2r, wr)


def _k4_body(gates_ref, gt_ref, scale_ref):
    e = pl.program_id(0)
    tb = pl.program_id(1)
    gates = gates_ref[...]
    iota8 = lax.broadcasted_iota(jnp.int32, (_BT, _NE), 1)
    gcol = jnp.sum(jnp.where(iota8 == e, gates, 0.0), axis=1, keepdims=True)
    grow = gt_ref[...].reshape(1, _S)
    icol = tb * _BT + lax.broadcasted_iota(jnp.int32, (_BT, 1), 0)
    irow = lax.broadcasted_iota(jnp.int32, (1, _S), 1)
    ahead = (grow > gcol) | ((grow == gcol) & (irow < icol))
    rank = jnp.sum(ahead.astype(jnp.float32), axis=1, keepdims=True)
    scale = jnp.where((rank < _CAP) & (gcol > 0.0), gcol, 0.0)
    scale_ref[...] = scale.reshape(1, _BT, 1)


def _k4(gates, gt3):
    return pl.pallas_call(
        _k4_body,
        grid=(_NE, _NTB),
        in_specs=[
            pl.BlockSpec((_BT, _NE), _nmap(1, None)),
            pl.BlockSpec((1, 1, _S), _nmap(0, None, None)),
        ],
        out_specs=pl.BlockSpec((1, _BT, 1), _nmap(0, 1, None)),
        out_shape=jax.ShapeDtypeStruct((_NE, _S, 1), jnp.float32),
    )(gates, gt3)


def _k5_body(h2_ref, sc_ref, w1_ref, w2_ref, out_ref):
    e = pl.program_id(0)
    fb = pl.program_id(1)
    scale = sc_ref[...].reshape(_S, 1)
    xs = h2_ref[...] * scale
    w1 = w1_ref[...].reshape(_BF, _D)
    hid = lax.dot_general(xs, w1, (((1,), (1,)), ((), ())),
                          preferred_element_type=jnp.float32)
    hid = jnp.maximum(hid, 0.0)
    w2 = w2_ref[...].reshape(_D, _BF)
    contrib = lax.dot_general(hid, w2, (((1,), (1,)), ((), ())),
                              preferred_element_type=jnp.float32)

    @pl.when((e == 0) & (fb == 0))
    def _init():
        out_ref[...] = contrib

    @pl.when((e > 0) | (fb > 0))
    def _acc():
        out_ref[...] += contrib


def _k5(h2, scale3, w1, w2):
    return pl.pallas_call(
        _k5_body,
        grid=(_NE, _NFB),
        in_specs=[
            pl.BlockSpec((_S, _D), _nmap(None, None)),
            pl.BlockSpec((1, _S, 1), _nmap(0, None, None)),
            pl.BlockSpec((1, _BF, _D), _nmap(0, 1, None)),
            pl.BlockSpec((1, _D, _BF), _nmap(0, None, 1)),
        ],
        out_specs=pl.BlockSpec((_S, _D), _nmap(None, None)),
        out_shape=jax.ShapeDtypeStruct((_S, _D), jnp.float32),
        compiler_params=pltpu.CompilerParams(
            dimension_semantics=("arbitrary", "arbitrary")),
    )(h2, scale3, w1, w2)


def _k6_body(x2_ref, yo_ref, out_ref):
    out_ref[...] = x2_ref[...] + yo_ref[...]


def _k6(x2, yout):
    return pl.pallas_call(
        _k6_body,
        grid=(_NTB,),
        in_specs=[pl.BlockSpec((_BT, _D), _nmap(0, None))] * 2,
        out_specs=pl.BlockSpec((_BT, _D), _nmap(0, None)),
        out_shape=jax.ShapeDtypeStruct((_S, _D), jnp.float32),
    )(x2, yout)


def _rope_tables():
    thetas = _BASE ** (-2 * (jnp.arange(_HALF, dtype=jnp.float32) / _HD))
    pos = jnp.arange(_S, dtype=jnp.float32)
    fr = jnp.outer(pos, thetas)
    cc = jnp.tile(jnp.cos(fr), (1, _N_HEADS))
    ss = jnp.tile(jnp.sin(fr), (1, _N_HEADS))
    return cc, ss


def _perm_qkv(Wqkv):
    h = jnp.arange(_N_HEADS)[:, None]
    i = jnp.arange(_HALF)[None, :]
    ev = (h * _HD + 2 * i).reshape(-1)
    od = (h * _HD + 2 * i + 1).reshape(-1)
    perm = jnp.concatenate([ev, od])
    wq = Wqkv[0:_D][perm]
    wk = Wqkv[_D:2 * _D][perm]
    wv = Wqkv[2 * _D:3 * _D]
    return jnp.concatenate([wq, wk, wv], axis=0)


def kernel(x, Wqkv, Wo, Wr, W1, W2, g1, g2):
    xf = x.reshape(_S, _D)
    cc, ss = _rope_tables()
    wp = _perm_qkv(Wqkv)
    q0, q1, k0, k1, v = _k1(xf, g1.reshape(1, _D), wp, cc, ss)
    y = _k2(q0, q1, k0, k1, v)
    x2, h2, gates = _k3(xf, y, Wo, g2.reshape(1, _D), Wr)
    gt3 = gates.T.reshape(_NE, 1, _S)
    scale3 = _k4(gates, gt3)
    yout = _k5(h2, scale3, W1, W2)
    out = _k6(x2, yout)
    return out.reshape(1, _S, _D)


# speedup vs baseline: 1.3556x; 1.1577x over previous
import math

import jax
import jax.numpy as jnp
from jax import lax
from jax.experimental import pallas as pl
from jax.experimental.pallas import tpu as pltpu
from jax.experimental.pallas import tpu_sc as plsc

_N_HEADS = 16
_BASE = 10000.0
_TOPK = 2
_CAP_FACTOR = 1.25
_EPS = 1e-6
_NE = 8
_D = 1024
_F = 4096
_S = 2048
_HD = _D // _N_HEADS
_HALF = _HD // 2
_DH = _D // 2
_CAP = max(math.ceil(_S * _TOPK * _CAP_FACTOR / _NE), 1)

_BT = 256
_NTB = _S // _BT
_HPB = 4
_NHB = _N_HEADS // _HPB
_BF = 512
_NFB = _F // _BF


def _nmap(*axes):
    def m(*pids):
        return tuple(pids[a] if isinstance(a, int) else 0 for a in axes)
    return m


def _k1_body(x_ref, g1_ref, w_ref, c_ref, s_ref,
             q0_ref, q1_ref, k0_ref, k1_ref, v_ref):
    x = x_ref[...]
    rms = lax.rsqrt(jnp.mean(x * x, axis=-1, keepdims=True) + _EPS)
    h = x * rms * g1_ref[...]
    qkv = lax.dot_general(h, w_ref[...], (((1,), (1,)), ((), ())),
                          preferred_element_type=jnp.float32)
    c = c_ref[...]
    s = s_ref[...]
    q0 = qkv[:, 0:_DH]
    q1 = qkv[:, _DH:_D]
    k0 = qkv[:, _D:_D + _DH]
    k1 = qkv[:, _D + _DH:2 * _D]
    q0_ref[...] = q0 * c - q1 * s
    q1_ref[...] = q0 * s + q1 * c
    k0_ref[...] = k0 * c - k1 * s
    k1_ref[...] = k0 * s + k1 * c
    v_ref[...] = qkv[:, 2 * _D:3 * _D]


def _k1(xf, g1r, wp, cc, ss):
    o = jax.ShapeDtypeStruct
    return pl.pallas_call(
        _k1_body,
        grid=(_NTB,),
        in_specs=[
            pl.BlockSpec((_BT, _D), _nmap(0, None)),
            pl.BlockSpec((1, _D), _nmap(None, None)),
            pl.BlockSpec((3 * _D, _D), _nmap(None, None)),
            pl.BlockSpec((_BT, _DH), _nmap(0, None)),
            pl.BlockSpec((_BT, _DH), _nmap(0, None)),
        ],
        out_specs=[pl.BlockSpec((_BT, _DH), _nmap(0, None))] * 4
        + [pl.BlockSpec((_BT, _D), _nmap(0, None))],
        out_shape=[o((_S, _DH), jnp.float32)] * 4 + [o((_S, _D), jnp.float32)],
    )(xf, g1r, wp, cc, ss)


def _k2_body(q0_ref, q1_ref, k0_ref, k1_ref, v_ref, y_ref):
    qb = pl.program_id(1)
    q0 = q0_ref[...]
    q1 = q1_ref[...]
    k0 = k0_ref[...]
    k1 = k1_ref[...]
    v = v_ref[...]
    row = qb * _BT + lax.broadcasted_iota(jnp.int32, (_BT, 1), 0)
    col = lax.broadcasted_iota(jnp.int32, (1, _S), 1)
    mask = col <= row
    dnt = (((1,), (1,)), ((), ()))
    dn = (((1,), (0,)), ((), ()))
    outs = []
    for hh in range(_HPB):
        hs = slice(hh * _HALF, (hh + 1) * _HALF)
        sc = (lax.dot_general(q0[:, hs], k0[:, hs], dnt,
                              preferred_element_type=jnp.float32)
              + lax.dot_general(q1[:, hs], k1[:, hs], dnt,
                                preferred_element_type=jnp.float32))
        sc = sc * (1.0 / math.sqrt(_HD))
        sc = jnp.where(mask, sc, -1e30)
        m = jnp.max(sc, axis=1, keepdims=True)
        p = jnp.exp(sc - m)
        p = p / jnp.sum(p, axis=1, keepdims=True)
        outs.append(lax.dot_general(p, v[:, hh * _HD:(hh + 1) * _HD], dn,
                                    preferred_element_type=jnp.float32))
    y_ref[...] = jnp.concatenate(outs, axis=1)


def _k2(q0, q1, k0, k1, v):
    hw = _HPB * _HALF
    vw = _HPB * _HD
    return pl.pallas_call(
        _k2_body,
        grid=(_NHB, _NTB),
        in_specs=[
            pl.BlockSpec((_BT, hw), _nmap(1, 0)),
            pl.BlockSpec((_BT, hw), _nmap(1, 0)),
            pl.BlockSpec((_S, hw), _nmap(None, 0)),
            pl.BlockSpec((_S, hw), _nmap(None, 0)),
            pl.BlockSpec((_S, vw), _nmap(None, 0)),
        ],
        out_specs=pl.BlockSpec((_BT, vw), _nmap(1, 0)),
        out_shape=jax.ShapeDtypeStruct((_S, _D), jnp.float32),
    )(q0, q1, k0, k1, v)


def _k3_body(x_ref, y_ref, wo_ref, g2_ref, wr_ref,
             x2_ref, h2_ref, gates_ref):
    x2 = x_ref[...] + lax.dot_general(
        y_ref[...], wo_ref[...], (((1,), (1,)), ((), ())),
        preferred_element_type=jnp.float32)
    x2_ref[...] = x2
    rms = lax.rsqrt(jnp.mean(x2 * x2, axis=-1, keepdims=True) + _EPS)
    h2 = x2 * rms * g2_ref[...]
    h2_ref[...] = h2
    logits = lax.dot_general(h2, wr_ref[...], (((1,), (1,)), ((), ())),
                             preferred_element_type=jnp.float32)
    lm = jnp.max(logits, axis=1, keepdims=True)
    el = jnp.exp(logits - lm)
    probs = el / jnp.sum(el, axis=1, keepdims=True)
    iota = lax.broadcasted_iota(jnp.int32, (_BT, _NE), 1)
    m1 = jnp.max(probs, axis=1, keepdims=True)
    e1 = jnp.min(jnp.where(probs == m1, iota, _NE), axis=1, keepdims=True)
    pm = jnp.where(iota == e1, -1.0, probs)
    m2 = jnp.max(pm, axis=1, keepdims=True)
    e2 = jnp.min(jnp.where(pm == m2, iota, _NE), axis=1, keepdims=True)
    gates_ref[...] = (jnp.where(iota == e1, m1, 0.0)
                      + jnp.where(iota == e2, m2, 0.0))


def _k3(xf, y, wo, g2r, wr):
    o = jax.ShapeDtypeStruct
    return pl.pallas_call(
        _k3_body,
        grid=(_NTB,),
        in_specs=[
            pl.BlockSpec((_BT, _D), _nmap(0, None)),
            pl.BlockSpec((_BT, _D), _nmap(0, None)),
            pl.BlockSpec((_D, _D), _nmap(None, None)),
            pl.BlockSpec((1, _D), _nmap(None, None)),
            pl.BlockSpec((_NE, _D), _nmap(None, None)),
        ],
        out_specs=[
            pl.BlockSpec((_BT, _D), _nmap(0, None)),
            pl.BlockSpec((_BT, _D), _nmap(0, None)),
            pl.BlockSpec((_BT, _NE), _nmap(0, None)),
        ],
        out_shape=[o((_S, _D), jnp.float32), o((_S, _D), jnp.float32),
                   o((_S, _NE), jnp.float32)],
    )(xf, y, wo, g2r, wr)


def _k4_body(gates_ref, gt_ref, gi1_ref, gi2_ref, gg1_ref, gg2_ref):
    tb = pl.program_id(0)
    gates = gates_ref[...]
    gt = gt_ref[...]
    icol = tb * _BT + lax.broadcasted_iota(jnp.int32, (_BT, 1), 0)
    irow = lax.broadcasted_iota(jnp.int32, (1, _S), 1)
    ranks = []
    for e in range(_NE):
        grow = gt[e:e + 1, :]
        gcol = gates[:, e:e + 1]
        ahead = (grow > gcol) | ((grow == gcol) & (irow < icol))
        ranks.append(jnp.sum(ahead.astype(jnp.float32), axis=1,
                             keepdims=True))
    rank = jnp.concatenate(ranks, axis=1).astype(jnp.int32)
    iota8 = lax.broadcasted_iota(jnp.int32, (_BT, _NE), 1)
    m = gates > 0.0
    e1 = jnp.min(jnp.where(m, iota8, _NE), axis=1, keepdims=True)
    e2 = jnp.max(jnp.where(m, iota8, -1), axis=1, keepdims=True)
    r1 = jnp.sum(jnp.where(iota8 == e1, rank, 0), axis=1, keepdims=True)
    r2 = jnp.sum(jnp.where(iota8 == e2, rank, 0), axis=1, keepdims=True)
    p1 = jnp.sum(jnp.where(iota8 == e1, gates, 0.0), axis=1, keepdims=True)
    p2 = jnp.sum(jnp.where(iota8 == e2, gates, 0.0), axis=1, keepdims=True)
    k1 = r1 < _CAP
    k2 = r2 < _CAP
    gi1_ref[...] = jnp.where(k1, e1 * _CAP + r1, 0)
    gi2_ref[...] = jnp.where(k2, e2 * _CAP + r2, 0)
    gg1_ref[...] = jnp.where(k1, p1, 0.0)
    gg2_ref[...] = jnp.where(k2, p2, 0.0)


def _k4(gates, gt2):
    o = jax.ShapeDtypeStruct
    return pl.pallas_call(
        _k4_body,
        grid=(_NTB,),
        in_specs=[
            pl.BlockSpec((_BT, _NE), _nmap(0, None)),
            pl.BlockSpec((_NE, _S), _nmap(None, None)),
        ],
        out_specs=[pl.BlockSpec((_BT, 1), _nmap(0, None))] * 4,
        out_shape=[o((_S, 1), jnp.int32), o((_S, 1), jnp.int32),
                   o((_S, 1), jnp.float32), o((_S, 1), jnp.float32)],
    )(gates, gt2)


_NW = 32
_RPW = (_NE * _CAP) // _NW
_GCH = 80
_TCH = 16


def _sca_body(gi1_hbm, gi2_hbm, gg1_hbm, gg2_hbm, h2_hbm, xsel_hbm,
              idx_v, gi1_v, gi2_v, gg1_v, gg2_v, rows_v, sem):
    wid = lax.axis_index("s") * 2 + lax.axis_index("c")
    lo = wid * _RPW
    z16 = jnp.zeros((_TCH,), jnp.int32)
    for i in range(_RPW // _TCH):
        idx_v[pl.ds(i * _TCH, _TCH)] = z16
    pltpu.sync_copy(gi1_hbm, gi1_v)
    pltpu.sync_copy(gi2_hbm, gi2_v)
    pltpu.sync_copy(gg1_hbm, gg1_v)
    pltpu.sync_copy(gg2_hbm, gg2_v)

    def chunk(ci, _):
        t = ci * _TCH + lax.iota(jnp.int32, _TCH)
        for gv, wv in ((gi1_v, gg1_v), (gi2_v, gg2_v)):
            g = gv[pl.ds(ci * _TCH, _TCH)]
            w = wv[pl.ds(ci * _TCH, _TCH)]
            msk = (w > 0.0) & (g >= lo) & (g < lo + _RPW)
            plsc.store_scatter(idx_v, [g - lo], t, mask=msk)
        return _

    lax.fori_loop(0, _S // _TCH, chunk, None)
    for j in range(_RPW // _GCH):
        pltpu.async_copy(h2_hbm.at[idx_v.at[pl.ds(j * _GCH, _GCH)]],
                         rows_v, sem).wait()
        pltpu.sync_copy(rows_v, xsel_hbm.at[pl.ds(lo + j * _GCH, _GCH)])


def _sc_gather_sel(gi1, gi2, gg1, gg2, h2):
    mesh = plsc.VectorSubcoreMesh(core_axis_name="c", subcore_axis_name="s")
    f = pl.kernel(
        _sca_body,
        out_type=jax.ShapeDtypeStruct((_NE * _CAP, _D), jnp.float32),
        mesh=mesh,
        scratch_types=[
            pltpu.VMEM((_RPW,), jnp.int32),
            pltpu.VMEM((_S,), jnp.int32),
            pltpu.VMEM((_S,), jnp.int32),
            pltpu.VMEM((_S,), jnp.float32),
            pltpu.VMEM((_S,), jnp.float32),
            pltpu.VMEM((_GCH, _D), jnp.float32),
            pltpu.SemaphoreType.DMA,
        ],
        compiler_params=pltpu.CompilerParams(needs_layout_passes=False),
    )
    return f(gi1, gi2, gg1, gg2, h2)


_TPW = _S // _NW
_G2CH = 32


def _scb_body(gi1_hbm, gi2_hbm, ysel_hbm, y1_hbm, y2_hbm, gi_v, rows_v, sem):
    wid = lax.axis_index("s") * 2 + lax.axis_index("c")
    base = wid * _TPW
    for g_hbm, o_hbm in ((gi1_hbm, y1_hbm), (gi2_hbm, y2_hbm)):
        for j in range(_TPW // _G2CH):
            off = base + j * _G2CH
            pltpu.sync_copy(g_hbm.at[pl.ds(off, _G2CH)], gi_v)
            pltpu.async_copy(ysel_hbm.at[gi_v], rows_v, sem).wait()
            pltpu.sync_copy(rows_v, o_hbm.at[pl.ds(off, _G2CH)])


def _sc_gather_out(gi1, gi2, ysel):
    mesh = plsc.VectorSubcoreMesh(core_axis_name="c", subcore_axis_name="s")
    o = jax.ShapeDtypeStruct((_S, _D), jnp.float32)
    f = pl.kernel(
        _scb_body,
        out_type=[o, o],
        mesh=mesh,
        scratch_types=[
            pltpu.VMEM((_G2CH,), jnp.int32),
            pltpu.VMEM((_G2CH, _D), jnp.float32),
            pltpu.SemaphoreType.DMA,
        ],
    )
    return f(gi1, gi2, ysel)


def _k5_body(xs_ref, w1_ref, w2_ref, out_ref):
    fb = pl.program_id(1)
    w1 = w1_ref[...].reshape(_BF, _D)
    hid = lax.dot_general(xs_ref[...], w1, (((1,), (1,)), ((), ())),
                          preferred_element_type=jnp.float32)
    hid = jnp.maximum(hid, 0.0)
    w2 = w2_ref[...].reshape(_D, _BF)
    contrib = lax.dot_general(hid, w2, (((1,), (1,)), ((), ())),
                              preferred_element_type=jnp.float32)

    @pl.when(fb == 0)
    def _init():
        out_ref[...] = contrib

    @pl.when(fb > 0)
    def _acc():
        out_ref[...] += contrib


def _k5(xsel, w1, w2):
    return pl.pallas_call(
        _k5_body,
        grid=(_NE, _NFB),
        in_specs=[
            pl.BlockSpec((_CAP, _D), _nmap(0, None)),
            pl.BlockSpec((1, _BF, _D), _nmap(0, 1, None)),
            pl.BlockSpec((1, _D, _BF), _nmap(0, None, 1)),
        ],
        out_specs=pl.BlockSpec((_CAP, _D), _nmap(0, None)),
        out_shape=jax.ShapeDtypeStruct((_NE * _CAP, _D), jnp.float32),
        compiler_params=pltpu.CompilerParams(
            dimension_semantics=("arbitrary", "arbitrary")),
    )(xsel, w1, w2)


def _k6_body(x2_ref, y1_ref, y2_ref, gg1_ref, gg2_ref, out_ref):
    out_ref[...] = (x2_ref[...]
                    + gg1_ref[...] * y1_ref[...]
                    + gg2_ref[...] * y2_ref[...])


def _k6(x2, y1, y2, gg1, gg2):
    return pl.pallas_call(
        _k6_body,
        grid=(_NTB,),
        in_specs=[pl.BlockSpec((_BT, _D), _nmap(0, None))] * 3
        + [pl.BlockSpec((_BT, 1), _nmap(0, None))] * 2,
        out_specs=pl.BlockSpec((_BT, _D), _nmap(0, None)),
        out_shape=jax.ShapeDtypeStruct((_S, _D), jnp.float32),
    )(x2, y1, y2, gg1, gg2)


def _rope_tables():
    thetas = _BASE ** (-2 * (jnp.arange(_HALF, dtype=jnp.float32) / _HD))
    pos = jnp.arange(_S, dtype=jnp.float32)
    fr = jnp.outer(pos, thetas)
    cc = jnp.tile(jnp.cos(fr), (1, _N_HEADS))
    ss = jnp.tile(jnp.sin(fr), (1, _N_HEADS))
    return cc, ss


def _perm_qkv(Wqkv):
    h = jnp.arange(_N_HEADS)[:, None]
    i = jnp.arange(_HALF)[None, :]
    ev = (h * _HD + 2 * i).reshape(-1)
    od = (h * _HD + 2 * i + 1).reshape(-1)
    perm = jnp.concatenate([ev, od])
    wq = Wqkv[0:_D][perm]
    wk = Wqkv[_D:2 * _D][perm]
    wv = Wqkv[2 * _D:3 * _D]
    return jnp.concatenate([wq, wk, wv], axis=0)


def kernel(x, Wqkv, Wo, Wr, W1, W2, g1, g2):
    xf = x.reshape(_S, _D)
    cc, ss = _rope_tables()
    wp = _perm_qkv(Wqkv)
    q0, q1, k0, k1, v = _k1(xf, g1.reshape(1, _D), wp, cc, ss)
    y = _k2(q0, q1, k0, k1, v)
    x2, h2, gates = _k3(xf, y, Wo, g2.reshape(1, _D), Wr)
    gi1, gi2, gg1, gg2 = _k4(gates, gates.T)
    gi1f = gi1.reshape(_S)
    gi2f = gi2.reshape(_S)
    xsel = _sc_gather_sel(gi1f, gi2f, gg1.reshape(_S), gg2.reshape(_S), h2)
    ysel = _k5(xsel, W1, W2)
    y1, y2 = _sc_gather_out(gi1f, gi2f, ysel)
    out = _k6(x2, y1, y2, gg1, gg2)
    return out.reshape(1, _S, _D)
